# baseline (speedup 1.0000x reference)
import jax
import jax.numpy as jnp
from jax import lax
from jax.experimental import pallas as pl
from jax.experimental.pallas import tpu as pltpu
from jax.experimental.pallas import tpu_sc as plsc

_OFFSET = 500000
_B = 16384
_NC, _NS, _L = 2, 16, 16
_NW = _NC * _NS
_BPW = _B // _NW
_NCHUNK = 4
_CROWS = _BPW // _NCHUNK
_CVEC = _CROWS // _L


def _body(x_hbm, tab_hbm, bias_hbm, out_hbm,
          x_v, iu_v, im_v, ru_v, rm_v, y_v, bias_v, sem):
    wid = lax.axis_index("s") * _NC + lax.axis_index("c")
    base = wid * _BPW

    hb = pltpu.async_copy(bias_hbm, bias_v, sem)
    hx = pltpu.async_copy(x_hbm.at[pl.ds(base * 2, 2 * _BPW)], x_v, sem)
    hx.wait()

    lanes = lax.iota(jnp.int32, _L)
    evens = (lanes * 2) & (_L - 1)
    odds = evens + 1
    lo_half = lanes < 8

    gathers = []
    for c in range(_NCHUNK):
        for v in range(_CVEC):
            j = c * _CVEC + v
            a = x_v[pl.ds(j * 2 * _L, _L)]
            b = x_v[pl.ds(j * 2 * _L + _L, _L)]
            u = jnp.where(lo_half,
                          a.at[evens].get(mode="promise_in_bounds"),
                          b.at[evens].get(mode="promise_in_bounds"))
            m = jnp.where(lo_half,
                          a.at[odds].get(mode="promise_in_bounds"),
                          b.at[odds].get(mode="promise_in_bounds"))
            iu_v[c, pl.ds(v * _L, _L)] = u
            im_v[c, pl.ds(v * _L, _L)] = m + _OFFSET
        gathers.append(pltpu.async_copy(tab_hbm.at[iu_v.at[c]], ru_v.at[c], sem))
        gathers.append(pltpu.async_copy(tab_hbm.at[im_v.at[c]], rm_v.at[c], sem))

    hb.wait()
    for g in gathers:
        g.wait()

    bias_vec = bias_v[...]
    for c in range(_NCHUNK):
        for v in range(_CVEC):
            j = c * _CVEC + v
            y_v[pl.ds(j * _L, _L)] = (ru_v[c, pl.ds(v * _L, _L)]
                                      + rm_v[c, pl.ds(v * _L, _L)]
                                      + bias_vec)

    pltpu.sync_copy(y_v, out_hbm.at[pl.ds(base, _BPW)])


def kernel(x, fc_weight, bias):
    mesh = plsc.VectorSubcoreMesh(core_axis_name="c", subcore_axis_name="s")
    k = pl.kernel(
        _body,
        mesh=mesh,
        out_type=jax.ShapeDtypeStruct((_B,), jnp.float32),
        scratch_types=[
            pltpu.VMEM((2 * _BPW,), jnp.int32),
            pltpu.VMEM((_NCHUNK, _CROWS), jnp.int32),
            pltpu.VMEM((_NCHUNK, _CROWS), jnp.int32),
            pltpu.VMEM((_NCHUNK, _CROWS), jnp.float32),
            pltpu.VMEM((_NCHUNK, _CROWS), jnp.float32),
            pltpu.VMEM((_BPW,), jnp.float32),
            pltpu.VMEM((_L,), jnp.float32),
            pltpu.SemaphoreType.DMA,
        ],
    )
    x_flat = x.reshape(-1).astype(jnp.int32)
    tab = fc_weight.reshape(-1)
    bias16 = jnp.broadcast_to(bias.astype(jnp.float32), (_L,))
    y = k(x_flat, tab, bias16)
    return y.reshape(_B, 1)


# speedup vs baseline: 1.0116x; 1.0116x over previous
import jax
import jax.numpy as jnp
from jax import lax
from jax.experimental import pallas as pl
from jax.experimental.pallas import tpu as pltpu
from jax.experimental.pallas import tpu_sc as plsc

_OFFSET = 500000
_B = 16384
_NC, _NS, _L = 2, 16, 16
_NW = _NC * _NS
_BPW = _B // _NW
_NCHUNK = 4
_CROWS = _BPW // _NCHUNK
_CVEC = _CROWS // _L


def _body(x_hbm, tab_hbm, bias_hbm, out_hbm,
          x_v, iu_v, im_v, ru_v, rm_v, y_v, bias_v,
          sem0, sem1, sem2, sem3, semo):
    wid = lax.axis_index("s") * _NC + lax.axis_index("c")
    base = wid * _BPW
    sems = [sem0, sem1, sem2, sem3]

    hb = pltpu.async_copy(bias_hbm, bias_v, semo)

    hx = []
    for c in range(_NCHUNK):
        hx.append(pltpu.async_copy(
            x_hbm.at[pl.ds(2 * (base + c * _CROWS), 2 * _CROWS)],
            x_v.at[c], sems[c]))

    lanes = lax.iota(jnp.int32, _L)
    evens = (lanes * 2) & (_L - 1)
    odds = evens + 1
    lo_half = lanes < 8

    gathers = []
    for c in range(_NCHUNK):
        hx[c].wait()
        for v in range(_CVEC):
            a = x_v[c, pl.ds(v * 2 * _L, _L)]
            b = x_v[c, pl.ds(v * 2 * _L + _L, _L)]
            u = jnp.where(lo_half,
                          a.at[evens].get(mode="promise_in_bounds"),
                          b.at[evens].get(mode="promise_in_bounds"))
            m = jnp.where(lo_half,
                          a.at[odds].get(mode="promise_in_bounds"),
                          b.at[odds].get(mode="promise_in_bounds"))
            iu_v[c, pl.ds(v * _L, _L)] = u
            im_v[c, pl.ds(v * _L, _L)] = m + _OFFSET
        gathers.append(pltpu.async_copy(
            tab_hbm.at[iu_v.at[c]], ru_v.at[c], sems[c]))
        gathers.append(pltpu.async_copy(
            tab_hbm.at[im_v.at[c]], rm_v.at[c], sems[c]))

    hb.wait()
    bias_vec = bias_v[...]

    ho = []
    for c in range(_NCHUNK):
        gathers[2 * c].wait()
        gathers[2 * c + 1].wait()
        for v in range(_CVEC):
            s = pl.ds(v * _L, _L)
            y_v[c, s] = ru_v[c, s] + rm_v[c, s] + bias_vec
        ho.append(pltpu.async_copy(
            y_v.at[c], out_hbm.at[pl.ds(base + c * _CROWS, _CROWS)], semo))
    for h in ho:
        h.wait()


def kernel(x, fc_weight, bias):
    mesh = plsc.VectorSubcoreMesh(core_axis_name="c", subcore_axis_name="s")
    k = pl.kernel(
        _body,
        mesh=mesh,
        out_type=jax.ShapeDtypeStruct((_B,), jnp.float32),
        scratch_types=[
            pltpu.VMEM((_NCHUNK, 2 * _CROWS), jnp.int32),
            pltpu.VMEM((_NCHUNK, _CROWS), jnp.int32),
            pltpu.VMEM((_NCHUNK, _CROWS), jnp.int32),
            pltpu.VMEM((_NCHUNK, _CROWS), jnp.float32),
            pltpu.VMEM((_NCHUNK, _CROWS), jnp.float32),
            pltpu.VMEM((_NCHUNK, _CROWS), jnp.float32),
            pltpu.VMEM((_L,), jnp.float32),
            pltpu.SemaphoreType.DMA,
            pltpu.SemaphoreType.DMA,
            pltpu.SemaphoreType.DMA,
            pltpu.SemaphoreType.DMA,
            pltpu.SemaphoreType.DMA,
        ],
    )
    x_flat = x.reshape(-1).astype(jnp.int32)
    tab = fc_weight.reshape(-1)
    bias16 = jnp.broadcast_to(bias.astype(jnp.float32), (_L,))
    y = k(x_flat, tab, bias16)
    return y.reshape(_B, 1)
